# Initial kernel scaffold; baseline (speedup 1.0000x reference)
import jax
import jax.numpy as jnp
from jax.experimental import pallas as pl


def kernel(x, edge_index, W1_rel, W1_root, b1, W2_rel, W2_root, b2):
    raise NotImplementedError("write your pallas kernel here")


# speedup vs baseline: 3.4865x; 3.4865x over previous
import functools

import jax
import jax.numpy as jnp
from jax import lax
from jax.experimental import pallas as pl
from jax.experimental.pallas import tpu as pltpu
from jax.experimental.pallas import tpu_sc as plsc

NS = 16
NC = 2
K = 80
HALF = 128


def _make_seg_sum(n, e, idx_mult, core_mult):
    eps = e // NS
    n_iter = eps // K
    zr = 128
    n_pad = -(-n // (NS * zr)) * NS * zr
    rps = n_pad // NS
    n_wb = rps // zr

    mesh = plsc.VectorSubcoreMesh(core_axis_name="c", subcore_axis_name="s")

    @functools.partial(
        pl.kernel,
        out_type=jax.ShapeDtypeStruct((NC, n_pad, HALF), jnp.float32),
        mesh=mesh,
        scratch_types=[
            pltpu.VMEM((K,), jnp.int32),
            pltpu.VMEM((K,), jnp.int32),
            pltpu.VMEM((K,), jnp.int32),
            pltpu.VMEM((K, HALF), jnp.float32),
            pltpu.VMEM((zr, HALF), jnp.float32),
            pltpu.VMEM_SHARED((n_pad, HALF), jnp.float32),
            pltpu.SemaphoreType.DMA,
        ],
    )
    def seg_sum(table_hbm, src_hbm, dst_hbm, out_hbm,
                src_v, dst_v, idx_v, rows_v, zbuf, acc_sh, sem):
        c = lax.axis_index("c")
        s = lax.axis_index("s")

        def zero_row(i, _):
            for jj in range(HALF // 16):
                zbuf[i, pl.ds(jj * 16, 16)] = jnp.zeros((16,), jnp.float32)
            return 0
        lax.fori_loop(0, zr, zero_row, 0)
        for r in range(n_wb):
            pltpu.sync_copy(zbuf, acc_sh.at[pl.ds(s * rps + r * zr, zr)])
        plsc.subcore_barrier()

        def body(k, _):
            base = s * eps + k * K
            pltpu.sync_copy(src_hbm.at[pl.ds(base, K)], src_v)
            pltpu.sync_copy(dst_hbm.at[pl.ds(base, K)], dst_v)
            for j in range(K // 16):
                v = src_v[pl.ds(j * 16, 16)]
                idx_v[pl.ds(j * 16, 16)] = v * idx_mult + c * core_mult
            pltpu.async_copy(table_hbm.at[idx_v], rows_v, sem).wait()
            pltpu.sync_copy(rows_v, acc_sh.at[dst_v], add=True)
            return 0
        lax.fori_loop(0, n_iter, body, 0)
        plsc.subcore_barrier()

        for r in range(n_wb):
            row = s * rps + r * zr
            pltpu.sync_copy(acc_sh.at[pl.ds(row, zr)], zbuf)
            pltpu.sync_copy(zbuf, out_hbm.at[c, pl.ds(row, zr)])

    return seg_sum


def _layer1_body(agg_ref, x_ref, wrel_ref, wroot_ref, b_ref, out_ref):
    a0 = agg_ref[0]
    a1 = agg_ref[1]
    wrel = wrel_ref[...]
    h = jnp.dot(a0, wrel[:HALF, :], preferred_element_type=jnp.float32)
    h += jnp.dot(a1, wrel[HALF:, :], preferred_element_type=jnp.float32)
    h += jnp.dot(x_ref[...], wroot_ref[...], preferred_element_type=jnp.float32)
    h += b_ref[...]
    t = jnp.maximum(h, 0.0)
    out_ref[0] = t[:, :HALF]
    out_ref[1] = t[:, HALF:]


def _layer2_body(agg_ref, t_ref, wrel_ref, wroot_ref, b_ref, out_ref):
    wrel = wrel_ref[...]
    wroot = wroot_ref[...]
    h = jnp.dot(agg_ref[0], wrel[:HALF, :], preferred_element_type=jnp.float32)
    h += jnp.dot(agg_ref[1], wrel[HALF:, :], preferred_element_type=jnp.float32)
    h += jnp.dot(t_ref[0], wroot[:HALF, :], preferred_element_type=jnp.float32)
    h += jnp.dot(t_ref[1], wroot[HALF:, :], preferred_element_type=jnp.float32)
    h += b_ref[...]
    out_ref[...] = h


def _tc_layer1(agg, x, wrel, wroot, b, bn):
    n, d = x.shape
    grid = (n // bn,)
    return pl.pallas_call(
        _layer1_body,
        grid=grid,
        in_specs=[
            pl.BlockSpec((2, bn, HALF), lambda i: (0, i, 0)),
            pl.BlockSpec((bn, d), lambda i: (i, 0)),
            pl.BlockSpec((d, d), lambda i: (0, 0)),
            pl.BlockSpec((d, d), lambda i: (0, 0)),
            pl.BlockSpec((1, d), lambda i: (0, 0)),
        ],
        out_specs=pl.BlockSpec((2, bn, HALF), lambda i: (0, i, 0)),
        out_shape=jax.ShapeDtypeStruct((2, n, HALF), jnp.float32),
    )(agg, x, wrel, wroot, b.reshape(1, d))


def _tc_layer2(agg, t_split, wrel, wroot, b, bn):
    n = agg.shape[1]
    d = 2 * HALF
    grid = (n // bn,)
    return pl.pallas_call(
        _layer2_body,
        grid=grid,
        in_specs=[
            pl.BlockSpec((2, bn, HALF), lambda i: (0, i, 0)),
            pl.BlockSpec((2, bn, HALF), lambda i: (0, i, 0)),
            pl.BlockSpec((d, d), lambda i: (0, 0)),
            pl.BlockSpec((d, d), lambda i: (0, 0)),
            pl.BlockSpec((1, d), lambda i: (0, 0)),
        ],
        out_specs=pl.BlockSpec((bn, d), lambda i: (i, 0)),
        out_shape=jax.ShapeDtypeStruct((n, d), jnp.float32),
    )(agg, t_split, wrel, wroot, b.reshape(1, d))


def kernel(x, edge_index, W1_rel, W1_root, b1, W2_rel, W2_root, b2):
    n, d = x.shape
    e = edge_index.shape[1]
    src = edge_index[0]
    dst = edge_index[1]
    bn = 1000

    x2 = x.reshape(2 * n, HALF)
    agg1 = _make_seg_sum(n, e, 2, 1)(x2, src, dst)[:, :n]
    t_split = _tc_layer1(agg1, x, W1_rel, W1_root, b1, bn)

    t2 = t_split.reshape(2 * n, HALF)
    agg2 = _make_seg_sum(n, e, 1, n)(t2, src, dst)[:, :n]
    return _tc_layer2(agg2, t_split, W2_rel, W2_root, b2, bn)


# speedup vs baseline: 6.1455x; 1.7626x over previous
import functools

import jax
import jax.numpy as jnp
from jax import lax
from jax.experimental import pallas as pl
from jax.experimental.pallas import tpu as pltpu
from jax.experimental.pallas import tpu_sc as plsc

NS = 16
NC = 2
K = 80
HALF = 128


def _make_seg_sum(n, e):
    eps = e // NS
    n_chunk = eps // K
    assert n_chunk % 2 == 1, "pipeline below assumes an odd chunk count"
    zr = K
    n_pad = -(-n // (NS * zr)) * NS * zr
    rps = n_pad // NS
    n_wb = rps // zr

    mesh = plsc.VectorSubcoreMesh(core_axis_name="c", subcore_axis_name="s")

    @functools.partial(
        pl.kernel,
        out_type=jax.ShapeDtypeStruct((NC, n_pad, HALF), jnp.float32),
        mesh=mesh,
        scratch_types=[
            pltpu.VMEM((n_chunk, K), jnp.int32),
            pltpu.VMEM((K,), jnp.int32),
            pltpu.VMEM((K,), jnp.int32),
            pltpu.VMEM((K, HALF), jnp.float32),
            pltpu.VMEM((K, HALF), jnp.float32),
            pltpu.VMEM_SHARED((n_pad, HALF), jnp.float32),
            pltpu.SemaphoreType.DMA,
            pltpu.SemaphoreType.DMA,
            pltpu.SemaphoreType.DMA,
            pltpu.SemaphoreType.DMA,
        ],
    )
    def seg_sum(table_hbm, src_hbm, dst_hbm, out_hbm,
                src_all, dst_a, dst_b, rows_a, rows_b, acc_sh,
                sem_a, sem_b, sem_da, sem_db):
        c = lax.axis_index("c")
        s = lax.axis_index("s")

        def zero_row(i, _):
            for jj in range(HALF // 16):
                rows_a[i, pl.ds(jj * 16, 16)] = jnp.zeros((16,), jnp.float32)
            return 0
        lax.fori_loop(0, zr, zero_row, 0)
        for r in range(n_wb):
            pltpu.sync_copy(rows_a, acc_sh.at[pl.ds(s * rps + r * zr, zr)])

        pltpu.sync_copy(src_hbm.at[s], src_all)
        pltpu.sync_copy(dst_hbm.at[s, 0], dst_a)

        def gather_start(k, buf, sem):
            return pltpu.async_copy(table_hbm.at[c].at[src_all.at[k]], buf, sem)

        def gather_wait(k, buf, sem):
            pltpu.make_async_copy(table_hbm.at[c].at[src_all.at[k]], buf, sem).wait()

        def dst_start(k, buf, sem):
            return pltpu.async_copy(dst_hbm.at[s, k], buf, sem)

        def dst_wait(k, buf, sem):
            pltpu.make_async_copy(dst_hbm.at[s, k], buf, sem).wait()

        def scatter(k, dbuf, buf):
            pltpu.sync_copy(buf, acc_sh.at[dbuf], add=True)

        gather_start(0, rows_a, sem_a)
        dst_start(1, dst_b, sem_db)
        plsc.subcore_barrier()

        def body(p, _):
            k = 2 * p
            gather_wait(k, rows_a, sem_a)
            gather_start(k + 1, rows_b, sem_b)
            scatter(k, dst_a, rows_a)
            dst_start(k + 2, dst_a, sem_da)
            gather_wait(k + 1, rows_b, sem_b)
            gather_start(k + 2, rows_a, sem_a)
            dst_wait(k + 1, dst_b, sem_db)
            scatter(k + 1, dst_b, rows_b)
            dst_start(k + 3, dst_b, sem_db)
            dst_wait(k + 2, dst_a, sem_da)
            return 0
        lax.fori_loop(0, (n_chunk - 1) // 2, body, 0)
        gather_wait(n_chunk - 1, rows_a, sem_a)
        scatter(n_chunk - 1, dst_a, rows_a)
        dst_wait(n_chunk, dst_b, sem_db)
        plsc.subcore_barrier()

        for r in range(n_wb):
            row = s * rps + r * zr
            pltpu.sync_copy(acc_sh.at[pl.ds(row, zr)], rows_a)
            pltpu.sync_copy(rows_a, out_hbm.at[c, pl.ds(row, zr)])

    return seg_sum


def _layer1_body(agg_ref, x_ref, wrel_ref, wroot_ref, b_ref, out_ref):
    a0 = agg_ref[0]
    a1 = agg_ref[1]
    wrel = wrel_ref[...]
    h = jnp.dot(a0, wrel[:HALF, :], preferred_element_type=jnp.float32)
    h += jnp.dot(a1, wrel[HALF:, :], preferred_element_type=jnp.float32)
    h += jnp.dot(x_ref[...], wroot_ref[...], preferred_element_type=jnp.float32)
    h += b_ref[...]
    t = jnp.maximum(h, 0.0)
    out_ref[0] = t[:, :HALF]
    out_ref[1] = t[:, HALF:]


def _layer2_body(agg_ref, t_ref, wrel_ref, wroot_ref, b_ref, out_ref):
    wrel = wrel_ref[...]
    wroot = wroot_ref[...]
    h = jnp.dot(agg_ref[0], wrel[:HALF, :], preferred_element_type=jnp.float32)
    h += jnp.dot(agg_ref[1], wrel[HALF:, :], preferred_element_type=jnp.float32)
    h += jnp.dot(t_ref[0], wroot[:HALF, :], preferred_element_type=jnp.float32)
    h += jnp.dot(t_ref[1], wroot[HALF:, :], preferred_element_type=jnp.float32)
    h += b_ref[...]
    out_ref[...] = h


def _tc_layer1(agg, x, wrel, wroot, b, bn):
    n, d = x.shape
    grid = (n // bn,)
    return pl.pallas_call(
        _layer1_body,
        grid=grid,
        in_specs=[
            pl.BlockSpec((2, bn, HALF), lambda i: (0, i, 0)),
            pl.BlockSpec((bn, d), lambda i: (i, 0)),
            pl.BlockSpec((d, d), lambda i: (0, 0)),
            pl.BlockSpec((d, d), lambda i: (0, 0)),
            pl.BlockSpec((1, d), lambda i: (0, 0)),
        ],
        out_specs=pl.BlockSpec((2, bn, HALF), lambda i: (0, i, 0)),
        out_shape=jax.ShapeDtypeStruct((2, n, HALF), jnp.float32),
    )(agg, x, wrel, wroot, b.reshape(1, d))


def _tc_layer2(agg, t_split, wrel, wroot, b, bn):
    n = agg.shape[1]
    d = 2 * HALF
    grid = (n // bn,)
    return pl.pallas_call(
        _layer2_body,
        grid=grid,
        in_specs=[
            pl.BlockSpec((2, bn, HALF), lambda i: (0, i, 0)),
            pl.BlockSpec((2, bn, HALF), lambda i: (0, i, 0)),
            pl.BlockSpec((d, d), lambda i: (0, 0)),
            pl.BlockSpec((d, d), lambda i: (0, 0)),
            pl.BlockSpec((1, d), lambda i: (0, 0)),
        ],
        out_specs=pl.BlockSpec((bn, d), lambda i: (i, 0)),
        out_shape=jax.ShapeDtypeStruct((n, d), jnp.float32),
    )(agg, t_split, wrel, wroot, b.reshape(1, d))


def kernel(x, edge_index, W1_rel, W1_root, b1, W2_rel, W2_root, b2):
    n, d = x.shape
    e = edge_index.shape[1]
    n_chunk = e // (NS * K)
    src = edge_index[0].reshape(NS, n_chunk, K)
    dst = jnp.pad(edge_index[1].reshape(NS, n_chunk, K), ((0, 0), (0, 1), (0, 0)))
    bn = 1000

    seg_sum = _make_seg_sum(n, e)

    x2 = x.reshape(n, 2, HALF).transpose(1, 0, 2)
    agg1 = seg_sum(x2, src, dst)[:, :n]
    t_split = _tc_layer1(agg1, x, W1_rel, W1_root, b1, bn)

    agg2 = seg_sum(t_split, src, dst)[:, :n]
    return _tc_layer2(agg2, t_split, W2_rel, W2_root, b2, bn)
